# baseline (speedup 1.0000x reference)
import functools

import jax
import jax.numpy as jnp
from jax import lax
from jax.experimental import pallas as pl
from jax.experimental.pallas import tpu as pltpu
from jax.experimental.pallas import tpu_sc as plsc

_F = 26
_V = 100000
_D = 64
_B = 16384

_NC = 2
_NS = 16
_NW = _NC * _NS

_LOOKUPS = _B * _F
_PER_W = _LOOKUPS // _NW
_CH = 128
_NCH = _PER_W // _CH
_K = 4
_NSTEP = _NCH // _K
_NBLK = _LOOKUPS // _CH
_LANES = 16

_VB = 2048


def _transpose_body(src_ref, dst_ref):
    ident = jnp.eye(_D, dtype=jnp.float32)
    dst_ref[0, :, 0:_D] = jax.lax.dot_general(
        src_ref[0],
        ident,
        (((0,), (0,)), ((), ())),
        precision=jax.lax.Precision.HIGHEST,
        preferred_element_type=jnp.float32,
    )


def _relayout(emb):
    embt = emb.transpose(0, 2, 1)
    grid = (_F, pl.cdiv(_V, _VB))
    padded = pl.pallas_call(
        _transpose_body,
        grid=grid,
        in_specs=[pl.BlockSpec((1, _D, _VB), lambda f, j: (f, 0, j))],
        out_specs=pl.BlockSpec((1, _VB, 2 * _D), lambda f, j: (f, j, 0)),
        out_shape=jax.ShapeDtypeStruct((_F, _V, 2 * _D), jnp.float32),
        compiler_params=pltpu.CompilerParams(
            dimension_semantics=("arbitrary", "arbitrary"),
        ),
    )(embt)
    return padded.reshape(_F * _V * 2, _D)


def _gather_body(x_hbm, tab_hbm, out_hbm, xv, rows0, rows1, sg0, sg1, sw0, sw1):
    wid = lax.axis_index("s") * _NC + lax.axis_index("c")
    blk_base = wid * _NCH

    pltpu.sync_copy(x_hbm.at[wid], xv)

    lane = lax.broadcasted_iota(jnp.int32, (_LANES,), 0)

    def add_offsets(r, carry):
        for c in range(_CH // _LANES):
            q0 = r * _CH + c * _LANES
            pos = lane + q0
            field = lax.rem(pos, _F)
            sl = pl.ds(c * _LANES, _LANES)
            xv[r, sl] = (xv[r, sl] + field * _V) * 2
        return carry

    lax.fori_loop(0, _NCH, add_offsets, 0)

    def fire(g, rows, sg):
        return [
            pltpu.async_copy(tab_hbm.at[xv.at[g * _K + kk]], rows.at[kk], sg)
            for kk in range(_K)
        ]

    def wait_wb(rows, sw):
        pltpu.make_async_copy(rows, out_hbm.at[pl.ds(0, _K)], sw).wait()

    def step(g, rows, sg, sw, first):
        if not first:
            wait_wb(rows, sw)
        for c in fire(g, rows, sg):
            c.wait()
        pltpu.async_copy(rows, out_hbm.at[pl.ds(blk_base + g * _K, _K)], sw)

    step(0, rows0, sg0, sw0, True)
    step(1, rows1, sg1, sw1, True)

    def pair(m, carry):
        g = 2 + 2 * m
        step(g, rows0, sg0, sw0, False)
        step(g + 1, rows1, sg1, sw1, False)
        return carry

    lax.fori_loop(0, (_NSTEP - 2) // 2, pair, 0)

    wait_wb(rows0, sw0)
    wait_wb(rows1, sw1)


def kernel(x, emb):
    xr = x.astype(jnp.int32).reshape(_NW, _NCH, _CH)
    tab = _relayout(emb)
    mesh = plsc.VectorSubcoreMesh(core_axis_name="c", subcore_axis_name="s")
    k = functools.partial(
        pl.kernel,
        mesh=mesh,
        out_type=jax.ShapeDtypeStruct((_NBLK, _CH, _D), jnp.float32),
        compiler_params=pltpu.CompilerParams(use_tc_tiling_on_sc=False),
        scratch_types=[
            pltpu.VMEM((_NCH, _CH), jnp.int32),
            pltpu.VMEM((_K, _CH, _D), jnp.float32),
            pltpu.VMEM((_K, _CH, _D), jnp.float32),
            pltpu.SemaphoreType.DMA,
            pltpu.SemaphoreType.DMA,
            pltpu.SemaphoreType.DMA,
            pltpu.SemaphoreType.DMA,
        ],
    )(_gather_body)
    out = k(xr, tab)
    return out.reshape(_B, _F * _D)


# speedup vs baseline: 1.2286x; 1.2286x over previous
import functools

import jax
import jax.numpy as jnp
from jax import lax
from jax.experimental import pallas as pl
from jax.experimental.pallas import tpu as pltpu
from jax.experimental.pallas import tpu_sc as plsc

_F = 26
_V = 100000
_D = 64
_B = 16384

_NC = 2
_NS = 16
_NW = _NC * _NS

_LOOKUPS = _B * _F
_PER_W = _LOOKUPS // _NW
_CH = 128
_NCH = _PER_W // _CH
_K = 4
_NSTEP = _NCH // _K
_NBLK = _LOOKUPS // _CH
_LANES = 16

_VB = 2048


def _transpose_body(src_ref, dst_ref):
    ident = jnp.eye(_D, dtype=jnp.float32)
    dst_ref[0, :, 0:_D] = jax.lax.dot_general(
        src_ref[0],
        ident,
        (((0,), (0,)), ((), ())),
        preferred_element_type=jnp.float32,
    )


def _relayout(emb):
    embt = emb.transpose(0, 2, 1)
    grid = (_F, pl.cdiv(_V, _VB))
    padded = pl.pallas_call(
        _transpose_body,
        grid=grid,
        in_specs=[pl.BlockSpec((1, _D, _VB), lambda f, j: (f, 0, j))],
        out_specs=pl.BlockSpec((1, _VB, 2 * _D), lambda f, j: (f, j, 0)),
        out_shape=jax.ShapeDtypeStruct((_F, _V, 2 * _D), jnp.float32),
        compiler_params=pltpu.CompilerParams(
            dimension_semantics=("arbitrary", "arbitrary"),
        ),
    )(embt)
    return padded.reshape(_F * _V * 2, _D)


def _gather_body(x_hbm, tab_hbm, out_hbm, xv, rows0, rows1, sg0, sg1, sw0, sw1):
    wid = lax.axis_index("s") * _NC + lax.axis_index("c")
    blk_base = wid * _NCH

    pltpu.sync_copy(x_hbm.at[wid], xv)

    lane = lax.broadcasted_iota(jnp.int32, (_LANES,), 0)

    def add_offsets(r, carry):
        for c in range(_CH // _LANES):
            q0 = r * _CH + c * _LANES
            pos = lane + q0
            field = lax.rem(pos, _F)
            sl = pl.ds(c * _LANES, _LANES)
            xv[r, sl] = (xv[r, sl] + field * _V) * 2
        return carry

    lax.fori_loop(0, _NCH, add_offsets, 0)

    def fire(g, rows, sg):
        return [
            pltpu.async_copy(tab_hbm.at[xv.at[g * _K + kk]], rows.at[kk], sg)
            for kk in range(_K)
        ]

    def wait_wb(rows, sw):
        pltpu.make_async_copy(rows, out_hbm.at[pl.ds(0, _K)], sw).wait()

    def step(g, rows, sg, sw, first):
        if not first:
            wait_wb(rows, sw)
        for c in fire(g, rows, sg):
            c.wait()
        pltpu.async_copy(rows, out_hbm.at[pl.ds(blk_base + g * _K, _K)], sw)

    step(0, rows0, sg0, sw0, True)
    step(1, rows1, sg1, sw1, True)

    def pair(m, carry):
        g = 2 + 2 * m
        step(g, rows0, sg0, sw0, False)
        step(g + 1, rows1, sg1, sw1, False)
        return carry

    lax.fori_loop(0, (_NSTEP - 2) // 2, pair, 0)

    wait_wb(rows0, sw0)
    wait_wb(rows1, sw1)


def kernel(x, emb):
    xr = x.astype(jnp.int32).reshape(_NW, _NCH, _CH)
    tab = _relayout(emb)
    mesh = plsc.VectorSubcoreMesh(core_axis_name="c", subcore_axis_name="s")
    k = functools.partial(
        pl.kernel,
        mesh=mesh,
        out_type=jax.ShapeDtypeStruct((_NBLK, _CH, _D), jnp.float32),
        compiler_params=pltpu.CompilerParams(use_tc_tiling_on_sc=False),
        scratch_types=[
            pltpu.VMEM((_NCH, _CH), jnp.int32),
            pltpu.VMEM((_K, _CH, _D), jnp.float32),
            pltpu.VMEM((_K, _CH, _D), jnp.float32),
            pltpu.SemaphoreType.DMA,
            pltpu.SemaphoreType.DMA,
            pltpu.SemaphoreType.DMA,
            pltpu.SemaphoreType.DMA,
        ],
    )(_gather_body)
    out = k(xr, tab)
    return out.reshape(_B, _F * _D)


# speedup vs baseline: 1.2799x; 1.0418x over previous
import functools

import jax
import jax.numpy as jnp
from jax import lax
from jax.experimental import pallas as pl
from jax.experimental.pallas import tpu as pltpu
from jax.experimental.pallas import tpu_sc as plsc

_F = 26
_V = 100000
_D = 64
_B = 16384

_NC = 2
_NS = 16
_NW = _NC * _NS

_LOOKUPS = _B * _F
_PER_W = _LOOKUPS // _NW
_CH = 128
_NCH = _PER_W // _CH
_K = 4
_NSTEP = _NCH // _K
_NBLK = _LOOKUPS // _CH
_LANES = 16

_VB = 2048


def _transpose_body(src_ref, dst_ref):
    dst_ref[0, :, 0:_D] = jnp.swapaxes(src_ref[0], 0, 1)


def _relayout(emb):
    embt = emb.transpose(0, 2, 1)
    grid = (_F, pl.cdiv(_V, _VB))
    padded = pl.pallas_call(
        _transpose_body,
        grid=grid,
        in_specs=[pl.BlockSpec((1, _D, _VB), lambda f, j: (f, 0, j))],
        out_specs=pl.BlockSpec((1, _VB, 2 * _D), lambda f, j: (f, j, 0)),
        out_shape=jax.ShapeDtypeStruct((_F, _V, 2 * _D), jnp.float32),
        compiler_params=pltpu.CompilerParams(
            dimension_semantics=("arbitrary", "arbitrary"),
        ),
    )(embt)
    return padded.reshape(_F * _V * 2, _D)


def _gather_body(x_hbm, tab_hbm, out_hbm, xv, rows0, rows1, sg0, sg1, sw0, sw1):
    wid = lax.axis_index("s") * _NC + lax.axis_index("c")
    blk_base = wid * _NCH

    pltpu.sync_copy(x_hbm.at[wid], xv)

    lane = lax.broadcasted_iota(jnp.int32, (_LANES,), 0)

    def add_offsets(r, carry):
        for c in range(_CH // _LANES):
            q0 = r * _CH + c * _LANES
            pos = lane + q0
            field = lax.rem(pos, _F)
            sl = pl.ds(c * _LANES, _LANES)
            xv[r, sl] = (xv[r, sl] + field * _V) * 2
        return carry

    lax.fori_loop(0, _NCH, add_offsets, 0)

    def fire(g, rows, sg):
        return [
            pltpu.async_copy(tab_hbm.at[xv.at[g * _K + kk]], rows.at[kk], sg)
            for kk in range(_K)
        ]

    def wait_wb(rows, sw):
        pltpu.make_async_copy(rows, out_hbm.at[pl.ds(0, _K)], sw).wait()

    def step(g, rows, sg, sw, first):
        if not first:
            wait_wb(rows, sw)
        for c in fire(g, rows, sg):
            c.wait()
        pltpu.async_copy(rows, out_hbm.at[pl.ds(blk_base + g * _K, _K)], sw)

    step(0, rows0, sg0, sw0, True)
    step(1, rows1, sg1, sw1, True)

    def pair(m, carry):
        g = 2 + 2 * m
        step(g, rows0, sg0, sw0, False)
        step(g + 1, rows1, sg1, sw1, False)
        return carry

    lax.fori_loop(0, (_NSTEP - 2) // 2, pair, 0)

    wait_wb(rows0, sw0)
    wait_wb(rows1, sw1)


def kernel(x, emb):
    xr = x.astype(jnp.int32).reshape(_NW, _NCH, _CH)
    tab = _relayout(emb)
    mesh = plsc.VectorSubcoreMesh(core_axis_name="c", subcore_axis_name="s")
    k = functools.partial(
        pl.kernel,
        mesh=mesh,
        out_type=jax.ShapeDtypeStruct((_NBLK, _CH, _D), jnp.float32),
        compiler_params=pltpu.CompilerParams(use_tc_tiling_on_sc=False),
        scratch_types=[
            pltpu.VMEM((_NCH, _CH), jnp.int32),
            pltpu.VMEM((_K, _CH, _D), jnp.float32),
            pltpu.VMEM((_K, _CH, _D), jnp.float32),
            pltpu.SemaphoreType.DMA,
            pltpu.SemaphoreType.DMA,
            pltpu.SemaphoreType.DMA,
            pltpu.SemaphoreType.DMA,
        ],
    )(_gather_body)
    out = k(xr, tab)
    return out.reshape(_B, _F * _D)
